# baseline (speedup 1.0000x reference)
import functools

import jax
import jax.numpy as jnp
from jax import lax
from jax.experimental import pallas as pl
from jax.experimental.pallas import tpu as pltpu
from jax.experimental.pallas import tpu_sc as plsc

_NUM_EMBEDDINGS = 1000000
_DIM = 64
_BATCH = 16384
_HIST = 50
_B = _BATCH * _HIST
_P = _B // 2
_NW = 32
_P_PER_W = _P // _NW
_CHUNK = 256
_N_CHUNKS = _P_PER_W // _CHUNK
_NBUF = 4
_N_UNITS = 2 * _N_CHUNKS
_N_GROUPS = _N_UNITS // _NBUF

_mesh = plsc.VectorSubcoreMesh(core_axis_name="c", subcore_axis_name="s")


@functools.partial(
    pl.kernel,
    mesh=_mesh,
    out_type=jax.ShapeDtypeStruct((_P, 2 * _DIM), jnp.float32),
    scratch_types=[
        [pltpu.VMEM((_CHUNK,), jnp.int32) for _ in range(_NBUF)],
        [pltpu.VMEM((_CHUNK, _DIM), jnp.float32) for _ in range(_NBUF)],
        [pltpu.SemaphoreType.DMA for _ in range(_NBUF)],
        [pltpu.SemaphoreType.DMA for _ in range(_NBUF)],
        [pltpu.SemaphoreType.DMA for _ in range(_NBUF)],
    ],
    compiler_params=pltpu.CompilerParams(use_tc_tiling_on_sc=False),
)
def _gather_rows(idx_e, idx_o, table_hbm, out_hbm, idxs, bufs, isems, gsems,
                 ssems):
    wid = lax.axis_index("s") * 2 + lax.axis_index("c")
    base = wid * _P_PER_W

    def i_copy(u, k):
        src = idx_e if k % 2 == 0 else idx_o
        j0 = base + (u // 2) * _CHUNK
        return pltpu.make_async_copy(
            src.at[pl.ds(j0, _CHUNK)], idxs[k], isems[k])

    def g_copy(u, k):
        del u
        return pltpu.make_async_copy(table_hbm.at[idxs[k]], bufs[k], gsems[k])

    def s_copy(u, k):
        j0 = base + (u // 2) * _CHUNK
        col = (k % 2) * _DIM
        return pltpu.make_async_copy(
            bufs[k], out_hbm.at[pl.ds(j0, _CHUNK), pl.ds(col, _DIM)],
            ssems[k])

    for b in range(_NBUF):
        i_copy(b, b).start()
    for b in range(2):
        i_copy(b, b).wait()
        g_copy(b, b).start()

    def step(u, k, do_sw, do_next, do_refill):
        g_copy(u, k).wait()
        s_copy(u, k).start()
        if do_next:
            if do_sw:
                s_copy(u - 2, (k - 2) % _NBUF).wait()
            i_copy(u + 2, (k + 2) % _NBUF).wait()
            g_copy(u + 2, (k + 2) % _NBUF).start()
        if do_refill:
            i_copy(u + _NBUF, k).start()

    for k in range(_NBUF):
        step(k, k, do_sw=(k >= 2), do_next=True, do_refill=True)

    def body(g, carry):
        u0 = g * _NBUF
        for k in range(_NBUF):
            step(u0 + k, k, do_sw=True, do_next=True, do_refill=True)
        return carry

    lax.fori_loop(1, _N_GROUPS - 1, body, 0)

    u0 = (_N_GROUPS - 1) * _NBUF
    for k in range(_NBUF):
        step(u0 + k, k, do_sw=(k < 2), do_next=(k < 2), do_refill=False)

    for u in range(_N_UNITS - 4, _N_UNITS):
        s_copy(u, u % _NBUF).wait()


def kernel(x, table):
    flat_idx = x.reshape(_B).astype(jnp.int32)
    out = _gather_rows(flat_idx[0::2], flat_idx[1::2], table)
    return out.reshape(_BATCH, _HIST, _DIM)


# speedup vs baseline: 1.0016x; 1.0016x over previous
import functools

import jax
import jax.numpy as jnp
from jax import lax
from jax.experimental import pallas as pl
from jax.experimental.pallas import tpu as pltpu
from jax.experimental.pallas import tpu_sc as plsc

_NUM_EMBEDDINGS = 1000000
_DIM = 64
_BATCH = 16384
_HIST = 50
_B = _BATCH * _HIST
_NW = 32
_BAT_PER_W = _BATCH // _NW
_CB = 8
_CHUNK = _CB * _HIST
_N_CHUNKS = _BAT_PER_W // _CB
_NBUF = 4
_N_GROUPS = _N_CHUNKS // _NBUF

_mesh = plsc.VectorSubcoreMesh(core_axis_name="c", subcore_axis_name="s")


@functools.partial(
    pl.kernel,
    mesh=_mesh,
    out_type=jax.ShapeDtypeStruct((_BATCH, _HIST, _DIM), jnp.float32),
    scratch_types=[
        [pltpu.VMEM((_CHUNK,), jnp.int32) for _ in range(_NBUF)],
        [pltpu.VMEM((_CHUNK, _DIM), jnp.float32) for _ in range(_NBUF)],
        [pltpu.SemaphoreType.DMA for _ in range(_NBUF)],
        [pltpu.SemaphoreType.DMA for _ in range(_NBUF)],
        [pltpu.SemaphoreType.DMA for _ in range(_NBUF)],
    ],
    compiler_params=pltpu.CompilerParams(use_tc_tiling_on_sc=False),
)
def _gather_rows(idx_hbm, table_hbm, out_hbm, idxs, bufs, isems, gsems, ssems):
    wid = lax.axis_index("s") * 2 + lax.axis_index("c")
    base_b = wid * _BAT_PER_W

    def i_copy(i, k):
        r0 = (base_b + i * _CB) * _HIST
        return pltpu.make_async_copy(
            idx_hbm.at[pl.ds(r0, _CHUNK)], idxs[k], isems[k])

    def g_copy(i, k):
        return pltpu.make_async_copy(table_hbm.at[idxs[k]], bufs[k], gsems[k])

    def s_copies(i, k):
        b0 = base_b + i * _CB
        return [
            pltpu.make_async_copy(
                bufs[k].at[pl.ds(j * _HIST, _HIST)], out_hbm.at[b0 + j],
                ssems[k])
            for j in range(_CB)
        ]

    for b in range(_NBUF):
        i_copy(b, b).start()
    for b in range(2):
        i_copy(b, b).wait()
        g_copy(b, b).start()

    def step(i, k, do_sw, do_next, do_refill):
        g_copy(i, k).wait()
        for c in s_copies(i, k):
            c.start()
        if do_next:
            if do_sw:
                for c in s_copies(i - 2, (k - 2) % _NBUF):
                    c.wait()
            i_copy(i + 2, (k + 2) % _NBUF).wait()
            g_copy(i + 2, (k + 2) % _NBUF).start()
        if do_refill:
            i_copy(i + _NBUF, k).start()

    for k in range(_NBUF):
        step(k, k, do_sw=(k >= 2), do_next=True, do_refill=True)

    def body(g, carry):
        i0 = g * _NBUF
        for k in range(_NBUF):
            step(i0 + k, k, do_sw=True, do_next=True, do_refill=True)
        return carry

    lax.fori_loop(1, _N_GROUPS - 1, body, 0)

    i0 = (_N_GROUPS - 1) * _NBUF
    for k in range(_NBUF):
        step(i0 + k, k, do_sw=(k < 2), do_next=(k < 2), do_refill=False)

    for i in range(_N_CHUNKS - 4, _N_CHUNKS):
        for c in s_copies(i, i % _NBUF):
            c.wait()


def kernel(x, table):
    flat_idx = x.reshape(_B).astype(jnp.int32)
    return _gather_rows(flat_idx, table)
